# baseline (speedup 1.0000x reference)
import functools

import jax
import jax.numpy as jnp
from jax.experimental import pallas as pl
from jax.experimental.pallas import tpu as pltpu

F32 = jnp.float32
IN_W = 263
NB = 1024
CD = 512
C = 512
NQ = 6
GROWTH = 3
DEPTH = 3
CPAD = 384

_PREC = jax.lax.Precision.HIGHEST
BF16 = jnp.bfloat16


def _dot_x(a, b):
    return jax.lax.dot_general(a, b, (((1,), (0,)), ((), ())),
                               preferred_element_type=F32, precision=_PREC)


def _dot(a, b):
    return jax.lax.dot_general(a.astype(BF16), b.astype(BF16),
                               (((1,), (0,)), ((), ())),
                               preferred_element_type=F32)


def _dot_t(a, b):
    return jax.lax.dot_general(a.astype(BF16), b.astype(BF16),
                               (((1,), (1,)), ((), ())),
                               preferred_element_type=F32)


def _shift_add(acc, y, s, seg, nseg):
    m, c = y.shape
    d = abs(s)
    if d == 0:
        return acc + y
    z = jnp.zeros((d, c), y.dtype)
    if s > 0:
        sh = jnp.concatenate([y[d:], z], axis=0)
        if nseg > 1:
            r = jax.lax.broadcasted_iota(jnp.int32, (m, 1), 0)
            sh = jnp.where(jax.lax.rem(r, seg) < seg - d, sh, 0.0)
    else:
        sh = jnp.concatenate([z, y[: m - d]], axis=0)
        if nseg > 1:
            r = jax.lax.broadcasted_iota(jnp.int32, (m, 1), 0)
            sh = jnp.where(jax.lax.rem(r, seg) >= d, sh, 0.0)
    return acc + sh


def _conv3_body(x_ref, w_ref, b_ref, o_ref, *, dil, bb, tin,
                pre_relu, post_relu, upsample):
    ci = x_ref.shape[-1]
    x = x_ref[...].reshape(bb * tin, ci)
    h = jnp.maximum(x, 0.0) if pre_relu else x
    if upsample:
        h = jnp.broadcast_to(h[:, None, :], (bb * tin, 2, ci))
        h = h.reshape(bb * tin * 2, ci)
    tl = tin * 2 if upsample else tin
    y0 = _dot(h, w_ref[0])
    y1 = _dot(h, w_ref[1])
    y2 = _dot(h, w_ref[2])
    acc = y1 + b_ref[...]
    acc = _shift_add(acc, y0, -dil, tl, bb)
    acc = _shift_add(acc, y2, dil, tl, bb)
    if post_relu:
        acc = jnp.maximum(acc, 0.0)
    o_ref[...] = acc.reshape(bb, tl, acc.shape[-1])


def _conv3(x, w, b, *, dil=1, bb=1, pre_relu=False, post_relu=False,
           upsample=False):
    n, tin, ci = x.shape
    co = w.shape[-1]
    tl = tin * 2 if upsample else tin
    return pl.pallas_call(
        functools.partial(_conv3_body, dil=dil, bb=bb, tin=tin,
                          pre_relu=pre_relu, post_relu=post_relu,
                          upsample=upsample),
        grid=(n // bb,),
        in_specs=[pl.BlockSpec((bb, tin, ci), lambda i: (i, 0, 0)),
                  pl.BlockSpec((3, ci, co), lambda i: (0, 0, 0)),
                  pl.BlockSpec((1, co), lambda i: (0, 0))],
        out_specs=pl.BlockSpec((bb, tl, co), lambda i: (i, 0, 0)),
        out_shape=jax.ShapeDtypeStruct((n, tl, co), F32),
        compiler_params=pltpu.CompilerParams(
            dimension_semantics=("arbitrary",)),
    )(x, w, b)


def _down4_body(x_ref, w_ref, b_ref, o_ref, *, bb, tin):
    c = x_ref.shape[-1]
    x = x_ref[...].reshape(bb * tin // 2, 2, c)
    ev = x[:, 0, :]
    od = x[:, 1, :]
    to = tin // 2
    acc = _dot(ev, w_ref[1]) + _dot(od, w_ref[2]) + b_ref[...]
    acc = _shift_add(acc, _dot(od, w_ref[0]), -1, to, bb)
    acc = _shift_add(acc, _dot(ev, w_ref[3]), 1, to, bb)
    o_ref[...] = acc.reshape(bb, to, c)


def _down4(x, w, b, *, bb=1):
    n, tin, c = x.shape
    to = tin // 2
    return pl.pallas_call(
        functools.partial(_down4_body, bb=bb, tin=tin),
        grid=(n // bb,),
        in_specs=[pl.BlockSpec((bb, tin, c), lambda i: (i, 0, 0)),
                  pl.BlockSpec((4, c, c), lambda i: (0, 0, 0)),
                  pl.BlockSpec((1, c), lambda i: (0, 0))],
        out_specs=pl.BlockSpec((bb, to, c), lambda i: (i, 0, 0)),
        out_shape=jax.ShapeDtypeStruct((n, to, c), F32),
        compiler_params=pltpu.CompilerParams(
            dimension_semantics=("arbitrary",)),
    )(x, w, b)


def _resblock_body(x_ref, w1_ref, b1_ref, w2_ref, b2_ref, o_ref, *, dil, bb,
                   tin):
    c = x_ref.shape[-1]
    x = x_ref[...].reshape(bb * tin, c)
    h = jnp.maximum(x, 0.0)
    y0 = _dot(h, w1_ref[0])
    y1 = _dot(h, w1_ref[1])
    y2 = _dot(h, w1_ref[2])
    acc = y1 + b1_ref[...]
    acc = _shift_add(acc, y0, -dil, tin, bb)
    acc = _shift_add(acc, y2, dil, tin, bb)
    h2 = jnp.maximum(acc, 0.0)
    out = x + _dot(h2, w2_ref[0]) + b2_ref[...]
    o_ref[...] = out.reshape(bb, tin, c)


def _resblock(x, w1, b1, w2, b2, *, dil, bb=1):
    n, tin, c = x.shape
    return pl.pallas_call(
        functools.partial(_resblock_body, dil=dil, bb=bb, tin=tin),
        grid=(n // bb,),
        in_specs=[pl.BlockSpec((bb, tin, c), lambda i: (i, 0, 0)),
                  pl.BlockSpec((3, c, c), lambda i: (0, 0, 0)),
                  pl.BlockSpec((1, c), lambda i: (0, 0)),
                  pl.BlockSpec((1, c, c), lambda i: (0, 0, 0)),
                  pl.BlockSpec((1, c), lambda i: (0, 0))],
        out_specs=pl.BlockSpec((bb, tin, c), lambda i: (i, 0, 0)),
        out_shape=jax.ShapeDtypeStruct((n, tin, c), F32),
        compiler_params=pltpu.CompilerParams(
            dimension_semantics=("arbitrary",)),
    )(x, w1, b1, w2, b2)


def _vq_body(flat_ref, cb_ref, q_ref, loss_ref, perp_ref, counts_ref, *,
             n_tiles, total_rows):
    i = pl.program_id(0)

    @pl.when(i == 0)
    def _init():
        loss_ref[...] = jnp.zeros((1, 1), F32)
        perp_ref[...] = jnp.zeros((1, 1), F32)
        counts_ref[...] = jnp.zeros_like(counts_ref)

    res = flat_ref[...]
    qout = jnp.zeros_like(res)
    total_sq = jnp.float32(0.0)
    lane = jax.lax.broadcasted_iota(jnp.int32, (res.shape[0], NB), 1)
    for q in range(NQ):
        cb = cb_ref[q]
        cbsq = jnp.sum(cb * cb, axis=1)
        xsq = jnp.sum(res * res, axis=1, keepdims=True)
        xc = _dot_t(res, cb)
        d = xsq - 2.0 * xc + cbsq[None, :]
        dmin = jnp.min(d, axis=1, keepdims=True)
        idx = jnp.min(jnp.where(d == dmin, lane, NB), axis=1)
        onehot = jnp.where(lane == idx[:, None], 1.0, 0.0).astype(F32)
        qvec = _dot_x(onehot, cb)
        diff = res - qvec
        total_sq = total_sq + jnp.sum(diff * diff)
        qout = qout + (res + (qvec - res))
        counts_ref[q, :] = counts_ref[q, :] + jnp.sum(onehot, axis=0)
        res = res - qvec
    q_ref[...] = qout
    loss_ref[...] = loss_ref[...] + total_sq

    @pl.when(i == n_tiles - 1)
    def _fin():
        loss_ref[...] = loss_ref[...] / (total_rows * CD)
        probs = counts_ref[...] / total_rows
        ent = -jnp.sum(probs * jnp.log(probs + 1e-10), axis=1)
        perp_ref[...] = jnp.mean(jnp.exp(ent)).reshape(1, 1)


def _vq(flat, cbs):
    rows = 128
    total = flat.shape[0]
    n_tiles = total // rows
    quant, loss, perp = pl.pallas_call(
        functools.partial(_vq_body, n_tiles=n_tiles, total_rows=total),
        grid=(n_tiles,),
        in_specs=[pl.BlockSpec((rows, CD), lambda i: (i, 0)),
                  pl.BlockSpec((NQ, NB, CD), lambda i: (0, 0, 0))],
        out_specs=[pl.BlockSpec((rows, CD), lambda i: (i, 0)),
                   pl.BlockSpec((1, 1), lambda i: (0, 0)),
                   pl.BlockSpec((1, 1), lambda i: (0, 0))],
        out_shape=[jax.ShapeDtypeStruct((total, CD), F32),
                   jax.ShapeDtypeStruct((1, 1), F32),
                   jax.ShapeDtypeStruct((1, 1), F32)],
        scratch_shapes=[pltpu.VMEM((NQ, NB), F32)],
        compiler_params=pltpu.CompilerParams(
            dimension_semantics=("arbitrary",)),
    )(flat, cbs)
    return quant, loss[0, 0], perp[0, 0]


def _prep(p):
    w = jnp.transpose(p["w"], (2, 1, 0))
    return w, p["b"][None, :]


def kernel(x, params):
    enc = params["enc"]
    dec = params["dec"]
    n, t, _ = x.shape

    xp = jnp.pad(x, ((0, 0), (0, 0), (0, CPAD - IN_W)))
    w, b = _prep(enc["conv_in"])
    w = jnp.pad(w, ((0, 0), (0, CPAD - IN_W), (0, 0)))
    h = _conv3(xp, w, b, post_relu=True)
    for st in enc["stages"]:
        wd, bd = _prep(st["down"])
        h = _down4(h, wd, bd)
        t //= 2
        for j, rp in enumerate(st["res"]):
            w1, b1 = _prep(rp["c1"])
            w2, b2 = _prep(rp["c2"])
            h = _resblock(h, w1, b1, w2, b2, dil=GROWTH ** j)
    wo, bo = _prep(enc["conv_out"])
    h = _conv3(h, wo, bo)

    flat = h.reshape(n * t, CD)
    quant, loss, perp = _vq(flat, params["codebooks"])
    h = quant.reshape(n, t, CD)

    wi, bi = _prep(dec["conv_in"])
    h = _conv3(h, wi, bi, post_relu=True)
    for st in dec["stages"]:
        for j, rp in enumerate(st["res"]):
            w1, b1 = _prep(rp["c1"])
            w2, b2 = _prep(rp["c2"])
            h = _resblock(h, w1, b1, w2, b2, dil=GROWTH ** (DEPTH - 1 - j))
        wu, bu = _prep(st["up_conv"])
        h = _conv3(h, wu, bu, upsample=True)
        t *= 2
    w1, b1 = _prep(dec["conv_out1"])
    h = _conv3(h, w1, b1, post_relu=True)
    w2, b2 = _prep(dec["conv_out2"])
    w2 = jnp.pad(w2, ((0, 0), (0, 0), (0, CPAD - IN_W)))
    b2 = jnp.pad(b2, ((0, 0), (0, CPAD - IN_W)))
    h = _conv3(h, w2, b2)
    rec = h[..., :IN_W].transpose(0, 2, 1)
    return rec, loss, perp
